# baseline (speedup 1.0000x reference)
import jax
import jax.numpy as jnp
from jax import lax
from jax.experimental import pallas as pl
from jax.experimental.pallas import tpu as pltpu
from jax.experimental.pallas import tpu_sc as plsc

NSEL = 300
B = 8
N = 5000
T = 256
C = 81
PC = 128
FLAT = N * PC
NW_PER_B = 4
QN = FLAT // NW_PER_B
CH = 8000
NCH = QN // CH
VPC = CH // 16
UNR = 10
NB1 = 1024
NB2 = 1024
NVW = QN // 16
GCAP = 512
FCAP = 512
ACAP = 2048
OSL = 304


def _mm_kernel(x_ref, w_ref, o_ref):
    s = jax.nn.sigmoid(x_ref[0])
    y = jnp.dot(s, w_ref[...], preferred_element_type=jnp.float32)
    col = lax.broadcasted_iota(jnp.int32, y.shape, 1)
    o_ref[0] = lax.bitcast_convert_type(
        jnp.where(col < C, y, -1.0), jnp.int32)


def _suffix_select(merged, target, lanes, nb):
    nv = nb // 16

    def step(t, st):
        found, jstar, above, carry = st
        j = nv - 1 - t
        vec = merged[pl.ds(j * 16, 16)]
        c = plsc.cumsum(vec)
        s = lax.reduce_sum_p.bind(vec, axes=(0,))
        suffix = carry + (s - c + vec)
        cond = (suffix >= target).astype(jnp.int32)
        pc = lax.reduce_sum_p.bind(cond, axes=(0,))
        has = pc > 0
        kstar = pc - 1
        oneh = lanes == kstar
        c_k = lax.reduce_sum_p.bind(jnp.where(oneh, c, 0), axes=(0,))
        v_k = lax.reduce_sum_p.bind(jnp.where(oneh, vec, 0), axes=(0,))
        s_j = carry + s - c_k + v_k
        take = jnp.logical_and(found == 0, has)
        jstar = jnp.where(take, j * 16 + kstar, jstar)
        above = jnp.where(take, s_j - v_k, above)
        found = jnp.where(has, 1, found)
        return (found, jstar, above, carry + s)

    st = lax.fori_loop(0, nv, step, (jnp.int32(0), jnp.int32(0),
                                     jnp.int32(0), jnp.int32(0)))
    return st[1], st[2]


def _sc_body(prob, boxes_in, scale_in,
             boxes_out, labels_out, scores_out,
             hist1, hist2, buf0, buf1, vmaxr, gidx, hitrows, merged, tmp,
             fin_v, fin_i, pv, pi, allv, alli,
             out_s, out_l, out_q, mo_s, mo_l, mo_q,
             boxes_v, bflat, scl,
             sh_h1, sh_h2, sh_fv, sh_fi, sh_os, sh_ol, sh_oq,
             sem0, sem1):
    cc = lax.axis_index("c")
    ss = lax.axis_index("s")
    b = cc * 4 + ss // NW_PER_B
    quarter = ss % NW_PER_B
    sg = (ss // NW_PER_B) * NW_PER_B
    lanes = lax.iota(jnp.int32, 16)
    ones = jnp.ones((16,), jnp.int32)
    zi = jnp.zeros((16,), jnp.int32)
    neg1 = jnp.full((16,), -1, jnp.int32)
    lane_nb1 = lanes * NB1
    lane_nb2 = lanes * NB2

    def zinit(i, _):
        hist1[pl.ds(i * 16, 16)] = zi
        return 0
    lax.fori_loop(0, 16 * NB1 // 16, zinit, 0)

    def zinit2(i, _):
        hist2[pl.ds(i * 16, 16)] = zi
        return 0
    lax.fori_loop(0, 16 * NB2 // 16, zinit2, 0)

    def finit(i, _):
        fin_v[pl.ds(i * 16, 16)] = neg1
        return 0
    lax.fori_loop(0, FCAP // 16, finit, 0)

    def ainit(i, _):
        allv[pl.ds(i * 16, 16)] = neg1
        return 0
    lax.fori_loop(0, ACAP // 16, ainit, 0)

    def oinit(i, _):
        out_s[pl.ds(i * 16, 16)] = zi
        out_l[pl.ds(i * 16, 16)] = zi
        out_q[pl.ds(i * 16, 16)] = zi
        return 0
    lax.fori_loop(0, OSL // 16, oinit, 0)

    qoff = quarter * QN
    pbase = b * FLAT + qoff

    def stream(vec_body, carry):
        cps = {}
        cps[0] = pltpu.async_copy(prob.at[pl.ds(pbase + 0 * CH, CH)],
                                  buf0, sem0)
        for k in range(NCH):
            buf = buf0 if k % 2 == 0 else buf1
            cps[k].wait()
            if k + 1 < NCH:
                cps[k + 1] = pltpu.async_copy(
                    prob.at[pl.ds(pbase + (k + 1) * CH, CH)],
                    buf1 if k % 2 == 0 else buf0,
                    sem1 if k % 2 == 0 else sem0)
            base = qoff + k * CH

            def step(i, cy):
                for u in range(UNR):
                    cy = vec_body(buf, base, i * UNR + u, cy)
                return cy
            carry = lax.fori_loop(0, VPC // UNR, step, carry)
        return carry

    def p1(buf, base, i, cy):
        v = buf[pl.ds(i * 16, 16)]
        key = jnp.minimum((v >> 21) & 0x7FF, NB1 - 1)
        mask = v >= 0
        plsc.addupdate_scatter(hist1, [lane_nb1 + key], ones, mask=mask)
        return cy
    stream(p1, jnp.int32(0))

    def lred1(i, _):
        acc = zi
        for l in range(16):
            acc = acc + hist1[pl.ds(l * NB1 + i * 16, 16)]
        merged[pl.ds(i * 16, 16)] = acc
        return 0
    lax.fori_loop(0, NB1 // 16, lred1, 0)
    pltpu.sync_copy(merged.at[pl.ds(0, NB1)], sh_h1.at[pl.ds(ss * NB1, NB1)])
    plsc.subcore_barrier()
    pltpu.sync_copy(sh_h1.at[pl.ds(sg * NB1, NB1)], merged.at[pl.ds(0, NB1)])
    for p in range(1, NW_PER_B):
        pltpu.sync_copy(sh_h1.at[pl.ds((sg + p) * NB1, NB1)], tmp.at[pl.ds(0, NB1)])

        def acc1(i, _):
            merged[pl.ds(i * 16, 16)] = (merged[pl.ds(i * 16, 16)]
                                         + tmp[pl.ds(i * 16, 16)])
            return 0
        lax.fori_loop(0, NB1 // 16, acc1, 0)

    j1, above1 = _suffix_select(merged, jnp.int32(NSEL), lanes, NB1)
    t1 = j1 << 21

    def p2(buf, base, i, cy):
        v = buf[pl.ds(i * 16, 16)]
        mask = (v >> 21) == j1
        key2 = (v >> 11) & (NB2 - 1)
        plsc.addupdate_scatter(hist2, [lane_nb2 + key2], ones, mask=mask)
        m = lax.reduce_max_p.bind(v, axes=(0,))
        vecid = (base - qoff) // 16 + i
        plsc.store_scatter(vmaxr, [jnp.full((16,), vecid, jnp.int32)],
                           jnp.full((16,), m, jnp.int32), mask=lanes == 0)
        return cy
    stream(p2, jnp.int32(0))

    def lred2(i, _):
        acc = zi
        for l in range(16):
            acc = acc + hist2[pl.ds(l * NB2 + i * 16, 16)]
        merged[pl.ds(i * 16, 16)] = acc
        return 0
    lax.fori_loop(0, NB2 // 16, lred2, 0)
    pltpu.sync_copy(merged, sh_h2.at[pl.ds(ss * NB2, NB2)])
    plsc.subcore_barrier()
    pltpu.sync_copy(sh_h2.at[pl.ds(sg * NB2, NB2)], merged)
    for p in range(1, NW_PER_B):
        pltpu.sync_copy(sh_h2.at[pl.ds((sg + p) * NB2, NB2)], tmp)

        def acc2(i, _):
            merged[pl.ds(i * 16, 16)] = (merged[pl.ds(i * 16, 16)]
                                         + tmp[pl.ds(i * 16, 16)])
            return 0
        lax.fori_loop(0, NB2 // 16, acc2, 0)

    j2, _ = _suffix_select(merged, jnp.int32(NSEL) - above1, lanes, NB2)
    t2 = t1 | (j2 << 11)

    pb16 = pbase // 16

    def ginit(i, _):
        gidx[pl.ds(i * 16, 16)] = jnp.full((16,), pb16, jnp.int32)
        return 0
    lax.fori_loop(0, GCAP // 16, ginit, 0)

    def scanv(i, cy):
        m = vmaxr[pl.ds(i * 16, 16)]
        mask = m >= t2
        cs = plsc.cumsum(mask.astype(jnp.int32))
        pos = jnp.minimum(cy + cs - 1, GCAP - 1)
        vid = pb16 + i * 16 + lanes
        plsc.store_scatter(gidx, [pos], vid, mask=mask)
        return cy + lax.reduce_sum_p.bind(jnp.where(mask, 1, 0), axes=(0,))
    nhit = lax.fori_loop(0, NVW // 16, scanv, jnp.int32(0))

    bq16 = b * (FLAT // 16)
    nblk = (nhit + 7) >> 3

    def fblk(bi, cy):
        vids = []
        cps = []
        for u in range(8):
            r = bi * 8 + u
            gvec = gidx[pl.ds((r >> 4) * 16, 16)]
            oneh = lanes == (r & 15)
            vid = lax.reduce_sum_p.bind(jnp.where(oneh, gvec, 0), axes=(0,))
            vids.append(vid)
            cps.append(pltpu.async_copy(
                prob.at[pl.ds(vid * 16, 16)],
                hitrows.at[pl.ds(u * 16, 16)], sem0))
        for cp in cps:
            cp.wait()
        for u in range(8):
            r = bi * 8 + u
            v = hitrows[pl.ds(u * 16, 16)]
            off = (vids[u] - bq16) * 16 + lanes
            mask = jnp.logical_and(v >= t2, r < nhit)
            cs = plsc.cumsum(mask.astype(jnp.int32))
            pos = jnp.minimum(cy + cs - 1, FCAP - 1)
            plsc.store_scatter(fin_v, [pos], v, mask=mask)
            plsc.store_scatter(fin_i, [pos], off, mask=mask)
            cy = cy + lax.reduce_sum_p.bind(jnp.where(mask, 1, 0), axes=(0,))
        return cy
    fcnt = lax.fori_loop(0, nblk, fblk, jnp.int32(0))

    pltpu.sync_copy(fin_v, sh_fv.at[pl.ds(ss * FCAP, FCAP)])
    pltpu.sync_copy(fin_i, sh_fi.at[pl.ds(ss * FCAP, FCAP)])
    plsc.subcore_barrier()

    mtot = jnp.int32(0)
    for p in range(NW_PER_B):
        pltpu.sync_copy(sh_fv.at[pl.ds((sg + p) * FCAP, FCAP)], pv)
        pltpu.sync_copy(sh_fi.at[pl.ds((sg + p) * FCAP, FCAP)], pi)

        def gcomp(i, cy):
            v = pv[pl.ds(i * 16, 16)]
            iv = pi[pl.ds(i * 16, 16)]
            mask = v >= 0
            cs = plsc.cumsum(mask.astype(jnp.int32))
            pos = jnp.minimum(cy + cs - 1, ACAP - 1)
            plsc.store_scatter(allv, [pos], v, mask=mask)
            plsc.store_scatter(alli, [pos], iv, mask=mask)
            return cy + lax.reduce_sum_p.bind(jnp.where(mask, 1, 0),
                                              axes=(0,))
        mtot = lax.fori_loop(0, FCAP // 16, gcomp, mtot)

    nj = (mtot + 15) >> 4

    def rank_outer(iv, _):
        @pl.when(iv * 16 < fcnt)
        def _():
            ov = fin_v[pl.ds(iv * 16, 16)]
            oi = fin_i[pl.ds(iv * 16, 16)]
            for ii in range(16):
                oneh = lanes == ii
                v_i = lax.reduce_sum_p.bind(jnp.where(oneh, ov, 0),
                                            axes=(0,))
                off_i = lax.reduce_sum_p.bind(jnp.where(oneh, oi, 0),
                                              axes=(0,))

                def jstep(jj, racc):
                    vj = allv[pl.ds(jj * 16, 16)]
                    ij = alli[pl.ds(jj * 16, 16)]
                    gt = vj > v_i
                    eq = jnp.logical_and(vj == v_i, ij < off_i)
                    return racc + jnp.where(jnp.logical_or(gt, eq), 1, 0)

                racc = lax.fori_loop(0, nj, jstep, zi)
                r_i = lax.reduce_sum_p.bind(racc, axes=(0,))

                @pl.when(jnp.logical_and(iv * 16 + ii < fcnt, r_i < NSEL))
                def _():
                    lane0 = lanes == 0
                    ridx = jnp.full((16,), r_i, jnp.int32)
                    plsc.store_scatter(out_s, [ridx],
                                       jnp.full((16,), v_i, jnp.int32),
                                       mask=lane0)
                    plsc.store_scatter(out_l, [ridx],
                                       jnp.full((16,), off_i & (PC - 1),
                                                jnp.int32), mask=lane0)
                    plsc.store_scatter(out_q, [ridx],
                                       jnp.full((16,), off_i >> 7,
                                                jnp.int32), mask=lane0)
        return 0
    lax.fori_loop(0, FCAP // 16, rank_outer, 0)

    pltpu.sync_copy(out_s, sh_os.at[pl.ds(ss * OSL, OSL)])
    pltpu.sync_copy(out_l, sh_ol.at[pl.ds(ss * OSL, OSL)])
    pltpu.sync_copy(out_q, sh_oq.at[pl.ds(ss * OSL, OSL)])
    plsc.subcore_barrier()

    @pl.when(quarter == 0)
    def _():
        pltpu.sync_copy(sh_os.at[pl.ds(sg * OSL, OSL)], mo_s)
        pltpu.sync_copy(sh_ol.at[pl.ds(sg * OSL, OSL)], mo_l)
        pltpu.sync_copy(sh_oq.at[pl.ds(sg * OSL, OSL)], mo_q)
        for p in range(1, NW_PER_B):
            pltpu.sync_copy(sh_os.at[pl.ds((sg + p) * OSL, OSL)], out_s)
            pltpu.sync_copy(sh_ol.at[pl.ds((sg + p) * OSL, OSL)], out_l)
            pltpu.sync_copy(sh_oq.at[pl.ds((sg + p) * OSL, OSL)], out_q)

            def omerge(i, _):
                sl = pl.ds(i * 16, 16)
                mo_s[sl] = mo_s[sl] + out_s[sl]
                mo_l[sl] = mo_l[sl] + out_l[sl]
                mo_q[sl] = mo_q[sl] + out_q[sl]
                return 0
            lax.fori_loop(0, OSL // 16, omerge, 0)

        pltpu.sync_copy(boxes_in.at[pl.ds(b * N * 4, N * 4)], boxes_v)
        pltpu.sync_copy(scale_in.at[pl.ds(b * 16, 16)], scl)
        sv = scl[...]
        s0 = lax.reduce_sum_p.bind(jnp.where(lanes == 0, sv, 0.0), axes=(0,))
        s1 = lax.reduce_sum_p.bind(jnp.where(lanes == 1, sv, 0.0), axes=(0,))
        s2 = lax.reduce_sum_p.bind(jnp.where(lanes == 2, sv, 0.0), axes=(0,))
        s3 = lax.reduce_sum_p.bind(jnp.where(lanes == 3, sv, 0.0), axes=(0,))

        def bgather(k, _):
            qv4 = mo_q[pl.ds(k * 16, 16)] * 4
            cx = plsc.load_gather(boxes_v, [qv4])
            cyv = plsc.load_gather(boxes_v, [qv4 + 1])
            w = plsc.load_gather(boxes_v, [qv4 + 2])
            h = plsc.load_gather(boxes_v, [qv4 + 3])
            x = (cx - 0.5 * w) * s0
            y = (cyv - 0.5 * h) * s1
            ws = w * s2
            hs = h * s3
            slot4 = (k * 16 + lanes) * 4
            plsc.store_scatter(bflat, [slot4], x)
            plsc.store_scatter(bflat, [slot4 + 1], y)
            plsc.store_scatter(bflat, [slot4 + 2], ws)
            plsc.store_scatter(bflat, [slot4 + 3], hs)
            return 0
        lax.fori_loop(0, OSL // 16, bgather, 0)

        pltpu.sync_copy(bflat, boxes_out.at[pl.ds(b * OSL * 4, OSL * 4)])
        pltpu.sync_copy(mo_s, scores_out.at[pl.ds(b * OSL, OSL)])
        pltpu.sync_copy(mo_l, labels_out.at[pl.ds(b * OSL, OSL)])


def _make_sc_kernel():
    mesh = plsc.VectorSubcoreMesh(core_axis_name="c", subcore_axis_name="s")
    return pl.kernel(
        _sc_body,
        out_type=(
            jax.ShapeDtypeStruct((B * OSL * 4,), jnp.float32),
            jax.ShapeDtypeStruct((B * OSL,), jnp.int32),
            jax.ShapeDtypeStruct((B * OSL,), jnp.int32),
        ),
        mesh=mesh,
        compiler_params=pltpu.CompilerParams(needs_layout_passes=False),
        scratch_types=[
            pltpu.VMEM((16 * NB1,), jnp.int32),
            pltpu.VMEM((16 * NB2,), jnp.int32),
            pltpu.VMEM((CH,), jnp.int32),
            pltpu.VMEM((CH,), jnp.int32),
            pltpu.VMEM((NVW,), jnp.int32),
            pltpu.VMEM((GCAP,), jnp.int32),
            pltpu.VMEM((128,), jnp.int32),
            pltpu.VMEM((NB2,), jnp.int32),
            pltpu.VMEM((NB2,), jnp.int32),
            pltpu.VMEM((FCAP,), jnp.int32),
            pltpu.VMEM((FCAP,), jnp.int32),
            pltpu.VMEM((FCAP,), jnp.int32),
            pltpu.VMEM((FCAP,), jnp.int32),
            pltpu.VMEM((ACAP,), jnp.int32),
            pltpu.VMEM((ACAP,), jnp.int32),
            pltpu.VMEM((OSL,), jnp.int32),
            pltpu.VMEM((OSL,), jnp.int32),
            pltpu.VMEM((OSL,), jnp.int32),
            pltpu.VMEM((OSL,), jnp.int32),
            pltpu.VMEM((OSL,), jnp.int32),
            pltpu.VMEM((OSL,), jnp.int32),
            pltpu.VMEM((N * 4,), jnp.float32),
            pltpu.VMEM((OSL * 4,), jnp.float32),
            pltpu.VMEM((16,), jnp.float32),
            pltpu.VMEM_SHARED((16 * NB1,), jnp.int32),
            pltpu.VMEM_SHARED((16 * NB2,), jnp.int32),
            pltpu.VMEM_SHARED((16 * FCAP,), jnp.int32),
            pltpu.VMEM_SHARED((16 * FCAP,), jnp.int32),
            pltpu.VMEM_SHARED((16 * OSL,), jnp.int32),
            pltpu.VMEM_SHARED((16 * OSL,), jnp.int32),
            pltpu.VMEM_SHARED((16 * OSL,), jnp.int32),
            pltpu.SemaphoreType.DMA,
            pltpu.SemaphoreType.DMA,
        ],
    )


def kernel(pred_logits, pred_boxes, target_sizes, positive_map):
    wT = jnp.pad(positive_map, ((0, PC - C), (0, 0))).T
    NBLK = 1000
    prob = pl.pallas_call(
        _mm_kernel,
        grid=(B, N // NBLK),
        in_specs=[
            pl.BlockSpec((1, NBLK, T), lambda i, j: (i, j, 0)),
            pl.BlockSpec((T, PC), lambda i, j: (0, 0)),
        ],
        out_specs=pl.BlockSpec((1, NBLK, PC), lambda i, j: (i, j, 0)),
        out_shape=jax.ShapeDtypeStruct((B, N, PC), jnp.int32),
    )(pred_logits, wT)

    img_h = target_sizes[:, 0]
    img_w = target_sizes[:, 1]
    scale = jnp.stack([img_w, img_h, img_w, img_h], axis=1).astype(jnp.float32)
    scale16 = jnp.pad(scale, ((0, 0), (0, 12)))

    boxes_flat, labels_p, scores_p = _make_sc_kernel()(
        prob.reshape(-1), pred_boxes.reshape(-1), scale16.reshape(-1))
    boxes = boxes_flat.reshape(B, OSL, 4)[:, :NSEL, :]
    labels = labels_p.reshape(B, OSL)[:, :NSEL]
    scores = lax.bitcast_convert_type(
        scores_p.reshape(B, OSL)[:, :NSEL], jnp.float32)
    return (boxes, labels, scores)


# speedup vs baseline: 1.0758x; 1.0758x over previous
import jax
import jax.numpy as jnp
from jax import lax
from jax.experimental import pallas as pl
from jax.experimental.pallas import tpu as pltpu
from jax.experimental.pallas import tpu_sc as plsc

NSEL = 300
B = 8
N = 5000
T = 256
C = 81
PC = 128
FLAT = N * PC
NW_PER_B = 4
QN = FLAT // NW_PER_B
CH = 8000
NCH = QN // CH
VPC = CH // 16
UNR = 10
NB1 = 1024
NB2 = 1024
NVW = QN // 16
GCAP = 512
FCAP = 512
ACAP = 2048
OSL = 304


def _mm_kernel(x_ref, w_ref, o_ref):
    s = jax.nn.sigmoid(x_ref[0])
    y = jnp.dot(s, w_ref[...], preferred_element_type=jnp.float32)
    col = lax.broadcasted_iota(jnp.int32, y.shape, 1)
    o_ref[0] = lax.bitcast_convert_type(
        jnp.where(col < C, y, -1.0), jnp.int32)


def _suffix_select(merged, target, lanes, nb):
    nv = nb // 16

    def step(t, st):
        found, jstar, above, carry = st
        j = nv - 1 - t
        vec = merged[pl.ds(j * 16, 16)]
        c = plsc.cumsum(vec)
        s = lax.reduce_sum_p.bind(vec, axes=(0,))
        suffix = carry + (s - c + vec)
        cond = (suffix >= target).astype(jnp.int32)
        pc = lax.reduce_sum_p.bind(cond, axes=(0,))
        has = pc > 0
        kstar = pc - 1
        oneh = lanes == kstar
        c_k = lax.reduce_sum_p.bind(jnp.where(oneh, c, 0), axes=(0,))
        v_k = lax.reduce_sum_p.bind(jnp.where(oneh, vec, 0), axes=(0,))
        s_j = carry + s - c_k + v_k
        take = jnp.logical_and(found == 0, has)
        jstar = jnp.where(take, j * 16 + kstar, jstar)
        above = jnp.where(take, s_j - v_k, above)
        found = jnp.where(has, 1, found)
        return (found, jstar, above, carry + s)

    st = lax.fori_loop(0, nv, step, (jnp.int32(0), jnp.int32(0),
                                     jnp.int32(0), jnp.int32(0)))
    return st[1], st[2]


def _sc_body(prob, boxes_in, scale_in,
             boxes_out, labels_out, scores_out,
             hist1, hist2, buf0, buf1, vmaxr, gidx, hitrows, merged, tmp,
             fin_v, fin_i, pv, pi, allv, alli,
             out_s, out_l, out_q, mo_s, mo_l, mo_q,
             boxes_v, bflat, scl,
             sh_h1, sh_h2, sh_fv, sh_fi, sh_os, sh_ol, sh_oq,
             sem0, sem1):
    cc = lax.axis_index("c")
    ss = lax.axis_index("s")
    b = cc * 4 + ss // NW_PER_B
    quarter = ss % NW_PER_B
    sg = (ss // NW_PER_B) * NW_PER_B
    lanes = lax.iota(jnp.int32, 16)
    ones = jnp.ones((16,), jnp.int32)
    zi = jnp.zeros((16,), jnp.int32)
    neg1 = jnp.full((16,), -1, jnp.int32)
    lane_nb1 = lanes * (NB1 + 1)
    lane_nb2 = lanes * (NB2 + 1)

    def zinit(i, _):
        hist1[pl.ds(i * 16, 16)] = zi
        return 0
    lax.fori_loop(0, (16 * (NB1 + 1) + 16) // 16, zinit, 0)

    def zinit2(i, _):
        hist2[pl.ds(i * 16, 16)] = zi
        return 0
    lax.fori_loop(0, (16 * (NB2 + 1) + 16) // 16, zinit2, 0)

    def finit(i, _):
        fin_v[pl.ds(i * 16, 16)] = neg1
        return 0
    lax.fori_loop(0, FCAP // 16, finit, 0)

    def ainit(i, _):
        allv[pl.ds(i * 16, 16)] = neg1
        return 0
    lax.fori_loop(0, ACAP // 16, ainit, 0)

    def oinit(i, _):
        out_s[pl.ds(i * 16, 16)] = zi
        out_l[pl.ds(i * 16, 16)] = zi
        out_q[pl.ds(i * 16, 16)] = zi
        return 0
    lax.fori_loop(0, OSL // 16, oinit, 0)

    qoff = quarter * QN
    pbase = b * FLAT + qoff

    def stream(vec_body, carry):
        cps = {}
        cps[0] = pltpu.async_copy(prob.at[pl.ds(pbase + 0 * CH, CH)],
                                  buf0, sem0)
        for k in range(NCH):
            buf = buf0 if k % 2 == 0 else buf1
            cps[k].wait()
            if k + 1 < NCH:
                cps[k + 1] = pltpu.async_copy(
                    prob.at[pl.ds(pbase + (k + 1) * CH, CH)],
                    buf1 if k % 2 == 0 else buf0,
                    sem1 if k % 2 == 0 else sem0)
            base = qoff + k * CH

            def step(i, cy):
                for u in range(UNR):
                    cy = vec_body(buf, base, i * UNR + u, cy)
                return cy
            carry = lax.fori_loop(0, VPC // UNR, step, carry)
        return carry

    def p1(buf, base, i, cy):
        v = buf[pl.ds(i * 16, 16)]
        key = jnp.minimum((v >> 21) & 0x7FF, NB1 - 1)
        mask = v >= 0
        plsc.addupdate_scatter(hist1, [lane_nb1 + key], ones, mask=mask)
        return cy
    stream(p1, jnp.int32(0))

    def lred1(i, _):
        acc = zi
        for l in range(16):
            acc = acc + hist1[pl.ds(l * (NB1 + 1) + i * 16, 16)]
        merged[pl.ds(i * 16, 16)] = acc
        return 0
    lax.fori_loop(0, NB1 // 16, lred1, 0)
    pltpu.sync_copy(merged.at[pl.ds(0, NB1)], sh_h1.at[pl.ds(ss * NB1, NB1)])
    plsc.subcore_barrier()
    pltpu.sync_copy(sh_h1.at[pl.ds(sg * NB1, NB1)], merged.at[pl.ds(0, NB1)])
    for p in range(1, NW_PER_B):
        pltpu.sync_copy(sh_h1.at[pl.ds((sg + p) * NB1, NB1)], tmp.at[pl.ds(0, NB1)])

        def acc1(i, _):
            merged[pl.ds(i * 16, 16)] = (merged[pl.ds(i * 16, 16)]
                                         + tmp[pl.ds(i * 16, 16)])
            return 0
        lax.fori_loop(0, NB1 // 16, acc1, 0)

    j1, above1 = _suffix_select(merged, jnp.int32(NSEL), lanes, NB1)
    t1 = j1 << 21

    def p2(buf, base, i, cy):
        v = buf[pl.ds(i * 16, 16)]
        mask = (v >> 21) == j1
        key2 = (v >> 11) & (NB2 - 1)
        plsc.addupdate_scatter(hist2, [lane_nb2 + key2], ones, mask=mask)
        m = lax.reduce_max_p.bind(v, axes=(0,))
        vecid = (base - qoff) // 16 + i
        plsc.store_scatter(vmaxr, [jnp.full((16,), vecid, jnp.int32)],
                           jnp.full((16,), m, jnp.int32), mask=lanes == 0)
        return cy
    stream(p2, jnp.int32(0))

    def lred2(i, _):
        acc = zi
        for l in range(16):
            acc = acc + hist2[pl.ds(l * (NB2 + 1) + i * 16, 16)]
        merged[pl.ds(i * 16, 16)] = acc
        return 0
    lax.fori_loop(0, NB2 // 16, lred2, 0)
    pltpu.sync_copy(merged, sh_h2.at[pl.ds(ss * NB2, NB2)])
    plsc.subcore_barrier()
    pltpu.sync_copy(sh_h2.at[pl.ds(sg * NB2, NB2)], merged)
    for p in range(1, NW_PER_B):
        pltpu.sync_copy(sh_h2.at[pl.ds((sg + p) * NB2, NB2)], tmp)

        def acc2(i, _):
            merged[pl.ds(i * 16, 16)] = (merged[pl.ds(i * 16, 16)]
                                         + tmp[pl.ds(i * 16, 16)])
            return 0
        lax.fori_loop(0, NB2 // 16, acc2, 0)

    j2, _ = _suffix_select(merged, jnp.int32(NSEL) - above1, lanes, NB2)
    t2 = t1 | (j2 << 11)

    pb16 = pbase // 16

    def ginit(i, _):
        gidx[pl.ds(i * 16, 16)] = jnp.full((16,), pb16, jnp.int32)
        return 0
    lax.fori_loop(0, GCAP // 16, ginit, 0)

    def scanv(i, cy):
        m = vmaxr[pl.ds(i * 16, 16)]
        mask = m >= t2
        cs = plsc.cumsum(mask.astype(jnp.int32))
        pos = jnp.minimum(cy + cs - 1, GCAP - 1)
        vid = pb16 + i * 16 + lanes
        plsc.store_scatter(gidx, [pos], vid, mask=mask)
        return cy + lax.reduce_sum_p.bind(jnp.where(mask, 1, 0), axes=(0,))
    nhit = lax.fori_loop(0, NVW // 16, scanv, jnp.int32(0))

    bq16 = b * (FLAT // 16)
    nblk = (nhit + 7) >> 3

    def fblk(bi, cy):
        vids = []
        cps = []
        for u in range(8):
            r = bi * 8 + u
            gvec = gidx[pl.ds((r >> 4) * 16, 16)]
            oneh = lanes == (r & 15)
            vid = lax.reduce_sum_p.bind(jnp.where(oneh, gvec, 0), axes=(0,))
            vids.append(vid)
            cps.append(pltpu.async_copy(
                prob.at[pl.ds(vid * 16, 16)],
                hitrows.at[pl.ds(u * 16, 16)], sem0))
        for cp in cps:
            cp.wait()
        for u in range(8):
            r = bi * 8 + u
            v = hitrows[pl.ds(u * 16, 16)]
            off = (vids[u] - bq16) * 16 + lanes
            mask = jnp.logical_and(v >= t2, r < nhit)
            cs = plsc.cumsum(mask.astype(jnp.int32))
            pos = jnp.minimum(cy + cs - 1, FCAP - 1)
            plsc.store_scatter(fin_v, [pos], v, mask=mask)
            plsc.store_scatter(fin_i, [pos], off, mask=mask)
            cy = cy + lax.reduce_sum_p.bind(jnp.where(mask, 1, 0), axes=(0,))
        return cy
    fcnt = lax.fori_loop(0, nblk, fblk, jnp.int32(0))

    pltpu.sync_copy(fin_v, sh_fv.at[pl.ds(ss * FCAP, FCAP)])
    pltpu.sync_copy(fin_i, sh_fi.at[pl.ds(ss * FCAP, FCAP)])
    plsc.subcore_barrier()

    mtot = jnp.int32(0)
    for p in range(NW_PER_B):
        pltpu.sync_copy(sh_fv.at[pl.ds((sg + p) * FCAP, FCAP)], pv)
        pltpu.sync_copy(sh_fi.at[pl.ds((sg + p) * FCAP, FCAP)], pi)

        def gcomp(i, cy):
            v = pv[pl.ds(i * 16, 16)]
            iv = pi[pl.ds(i * 16, 16)]
            mask = v >= 0
            cs = plsc.cumsum(mask.astype(jnp.int32))
            pos = jnp.minimum(cy + cs - 1, ACAP - 1)
            plsc.store_scatter(allv, [pos], v, mask=mask)
            plsc.store_scatter(alli, [pos], iv, mask=mask)
            return cy + lax.reduce_sum_p.bind(jnp.where(mask, 1, 0),
                                              axes=(0,))
        mtot = lax.fori_loop(0, FCAP // 16, gcomp, mtot)

    nj = (mtot + 15) >> 4

    def rank_outer(iv, _):
        @pl.when(iv * 16 < fcnt)
        def _():
            ov = fin_v[pl.ds(iv * 16, 16)]
            oi = fin_i[pl.ds(iv * 16, 16)]
            for ii in range(16):
                oneh = lanes == ii
                v_i = lax.reduce_sum_p.bind(jnp.where(oneh, ov, 0),
                                            axes=(0,))
                off_i = lax.reduce_sum_p.bind(jnp.where(oneh, oi, 0),
                                              axes=(0,))

                def jstep(jj, racc):
                    vj = allv[pl.ds(jj * 16, 16)]
                    ij = alli[pl.ds(jj * 16, 16)]
                    gt = vj > v_i
                    eq = jnp.logical_and(vj == v_i, ij < off_i)
                    return racc + jnp.where(jnp.logical_or(gt, eq), 1, 0)

                racc = lax.fori_loop(0, nj, jstep, zi)
                r_i = lax.reduce_sum_p.bind(racc, axes=(0,))

                @pl.when(jnp.logical_and(iv * 16 + ii < fcnt, r_i < NSEL))
                def _():
                    lane0 = lanes == 0
                    ridx = jnp.full((16,), r_i, jnp.int32)
                    plsc.store_scatter(out_s, [ridx],
                                       jnp.full((16,), v_i, jnp.int32),
                                       mask=lane0)
                    plsc.store_scatter(out_l, [ridx],
                                       jnp.full((16,), off_i & (PC - 1),
                                                jnp.int32), mask=lane0)
                    plsc.store_scatter(out_q, [ridx],
                                       jnp.full((16,), off_i >> 7,
                                                jnp.int32), mask=lane0)
        return 0
    lax.fori_loop(0, FCAP // 16, rank_outer, 0)

    pltpu.sync_copy(out_s, sh_os.at[pl.ds(ss * OSL, OSL)])
    pltpu.sync_copy(out_l, sh_ol.at[pl.ds(ss * OSL, OSL)])
    pltpu.sync_copy(out_q, sh_oq.at[pl.ds(ss * OSL, OSL)])
    plsc.subcore_barrier()

    @pl.when(quarter == 0)
    def _():
        pltpu.sync_copy(sh_os.at[pl.ds(sg * OSL, OSL)], mo_s)
        pltpu.sync_copy(sh_ol.at[pl.ds(sg * OSL, OSL)], mo_l)
        pltpu.sync_copy(sh_oq.at[pl.ds(sg * OSL, OSL)], mo_q)
        for p in range(1, NW_PER_B):
            pltpu.sync_copy(sh_os.at[pl.ds((sg + p) * OSL, OSL)], out_s)
            pltpu.sync_copy(sh_ol.at[pl.ds((sg + p) * OSL, OSL)], out_l)
            pltpu.sync_copy(sh_oq.at[pl.ds((sg + p) * OSL, OSL)], out_q)

            def omerge(i, _):
                sl = pl.ds(i * 16, 16)
                mo_s[sl] = mo_s[sl] + out_s[sl]
                mo_l[sl] = mo_l[sl] + out_l[sl]
                mo_q[sl] = mo_q[sl] + out_q[sl]
                return 0
            lax.fori_loop(0, OSL // 16, omerge, 0)

        pltpu.sync_copy(boxes_in.at[pl.ds(b * N * 4, N * 4)], boxes_v)
        pltpu.sync_copy(scale_in.at[pl.ds(b * 16, 16)], scl)
        sv = scl[...]
        s0 = lax.reduce_sum_p.bind(jnp.where(lanes == 0, sv, 0.0), axes=(0,))
        s1 = lax.reduce_sum_p.bind(jnp.where(lanes == 1, sv, 0.0), axes=(0,))
        s2 = lax.reduce_sum_p.bind(jnp.where(lanes == 2, sv, 0.0), axes=(0,))
        s3 = lax.reduce_sum_p.bind(jnp.where(lanes == 3, sv, 0.0), axes=(0,))

        def bgather(k, _):
            qv4 = mo_q[pl.ds(k * 16, 16)] * 4
            cx = plsc.load_gather(boxes_v, [qv4])
            cyv = plsc.load_gather(boxes_v, [qv4 + 1])
            w = plsc.load_gather(boxes_v, [qv4 + 2])
            h = plsc.load_gather(boxes_v, [qv4 + 3])
            x = (cx - 0.5 * w) * s0
            y = (cyv - 0.5 * h) * s1
            ws = w * s2
            hs = h * s3
            slot4 = (k * 16 + lanes) * 4
            plsc.store_scatter(bflat, [slot4], x)
            plsc.store_scatter(bflat, [slot4 + 1], y)
            plsc.store_scatter(bflat, [slot4 + 2], ws)
            plsc.store_scatter(bflat, [slot4 + 3], hs)
            return 0
        lax.fori_loop(0, OSL // 16, bgather, 0)

        pltpu.sync_copy(bflat, boxes_out.at[pl.ds(b * OSL * 4, OSL * 4)])
        pltpu.sync_copy(mo_s, scores_out.at[pl.ds(b * OSL, OSL)])
        pltpu.sync_copy(mo_l, labels_out.at[pl.ds(b * OSL, OSL)])


def _make_sc_kernel():
    mesh = plsc.VectorSubcoreMesh(core_axis_name="c", subcore_axis_name="s")
    return pl.kernel(
        _sc_body,
        out_type=(
            jax.ShapeDtypeStruct((B * OSL * 4,), jnp.float32),
            jax.ShapeDtypeStruct((B * OSL,), jnp.int32),
            jax.ShapeDtypeStruct((B * OSL,), jnp.int32),
        ),
        mesh=mesh,
        compiler_params=pltpu.CompilerParams(needs_layout_passes=False),
        scratch_types=[
            pltpu.VMEM((16 * (NB1 + 1) + 16,), jnp.int32),
            pltpu.VMEM((16 * (NB2 + 1) + 16,), jnp.int32),
            pltpu.VMEM((CH,), jnp.int32),
            pltpu.VMEM((CH,), jnp.int32),
            pltpu.VMEM((NVW,), jnp.int32),
            pltpu.VMEM((GCAP,), jnp.int32),
            pltpu.VMEM((128,), jnp.int32),
            pltpu.VMEM((NB2,), jnp.int32),
            pltpu.VMEM((NB2,), jnp.int32),
            pltpu.VMEM((FCAP,), jnp.int32),
            pltpu.VMEM((FCAP,), jnp.int32),
            pltpu.VMEM((FCAP,), jnp.int32),
            pltpu.VMEM((FCAP,), jnp.int32),
            pltpu.VMEM((ACAP,), jnp.int32),
            pltpu.VMEM((ACAP,), jnp.int32),
            pltpu.VMEM((OSL,), jnp.int32),
            pltpu.VMEM((OSL,), jnp.int32),
            pltpu.VMEM((OSL,), jnp.int32),
            pltpu.VMEM((OSL,), jnp.int32),
            pltpu.VMEM((OSL,), jnp.int32),
            pltpu.VMEM((OSL,), jnp.int32),
            pltpu.VMEM((N * 4,), jnp.float32),
            pltpu.VMEM((OSL * 4,), jnp.float32),
            pltpu.VMEM((16,), jnp.float32),
            pltpu.VMEM_SHARED((16 * NB1,), jnp.int32),
            pltpu.VMEM_SHARED((16 * NB2,), jnp.int32),
            pltpu.VMEM_SHARED((16 * FCAP,), jnp.int32),
            pltpu.VMEM_SHARED((16 * FCAP,), jnp.int32),
            pltpu.VMEM_SHARED((16 * OSL,), jnp.int32),
            pltpu.VMEM_SHARED((16 * OSL,), jnp.int32),
            pltpu.VMEM_SHARED((16 * OSL,), jnp.int32),
            pltpu.SemaphoreType.DMA,
            pltpu.SemaphoreType.DMA,
        ],
    )


def kernel(pred_logits, pred_boxes, target_sizes, positive_map):
    wT = jnp.pad(positive_map, ((0, PC - C), (0, 0))).T
    NBLK = 1000
    prob = pl.pallas_call(
        _mm_kernel,
        grid=(B, N // NBLK),
        in_specs=[
            pl.BlockSpec((1, NBLK, T), lambda i, j: (i, j, 0)),
            pl.BlockSpec((T, PC), lambda i, j: (0, 0)),
        ],
        out_specs=pl.BlockSpec((1, NBLK, PC), lambda i, j: (i, j, 0)),
        out_shape=jax.ShapeDtypeStruct((B, N, PC), jnp.int32),
    )(pred_logits, wT)

    img_h = target_sizes[:, 0]
    img_w = target_sizes[:, 1]
    scale = jnp.stack([img_w, img_h, img_w, img_h], axis=1).astype(jnp.float32)
    scale16 = jnp.pad(scale, ((0, 0), (0, 12)))

    boxes_flat, labels_p, scores_p = _make_sc_kernel()(
        prob.reshape(-1), pred_boxes.reshape(-1), scale16.reshape(-1))
    boxes = boxes_flat.reshape(B, OSL, 4)[:, :NSEL, :]
    labels = labels_p.reshape(B, OSL)[:, :NSEL]
    scores = lax.bitcast_convert_type(
        scores_p.reshape(B, OSL)[:, :NSEL], jnp.float32)
    return (boxes, labels, scores)


# speedup vs baseline: 1.3698x; 1.2734x over previous
import jax
import jax.numpy as jnp
from jax import lax
from jax.experimental import pallas as pl
from jax.experimental.pallas import tpu as pltpu
from jax.experimental.pallas import tpu_sc as plsc

NSEL = 300
B = 8
N = 5000
T = 256
C = 81
PC = 128
FLAT = N * PC
NW_PER_B = 4
QN = FLAT // NW_PER_B
CH = 8000
NCH = QN // CH
VPC = CH // 16
UNR = 10
NB1 = 1024
NB2 = 1024
NVG = QN // 160
GCAP = 512
FCAP = 512
ACAP = 2048
OSL = 304


def _mm_kernel(x_ref, w_ref, o_ref):
    s = jax.nn.sigmoid(x_ref[0])
    y = jnp.dot(s, w_ref[...], preferred_element_type=jnp.float32)
    col = lax.broadcasted_iota(jnp.int32, y.shape, 1)
    o_ref[0] = lax.bitcast_convert_type(
        jnp.where(col < C, y, -1.0), jnp.int32)


def _suffix_select(merged, target, lanes, nb):
    nv = nb // 16

    def step(t, st):
        found, jstar, above, carry = st
        j = nv - 1 - t
        vec = merged[pl.ds(j * 16, 16)]
        c = plsc.cumsum(vec)
        s = lax.reduce_sum_p.bind(vec, axes=(0,))
        suffix = carry + (s - c + vec)
        cond = (suffix >= target).astype(jnp.int32)
        pc = lax.reduce_sum_p.bind(cond, axes=(0,))
        has = pc > 0
        kstar = pc - 1
        oneh = lanes == kstar
        c_k = lax.reduce_sum_p.bind(jnp.where(oneh, c, 0), axes=(0,))
        v_k = lax.reduce_sum_p.bind(jnp.where(oneh, vec, 0), axes=(0,))
        s_j = carry + s - c_k + v_k
        take = jnp.logical_and(found == 0, has)
        jstar = jnp.where(take, j * 16 + kstar, jstar)
        above = jnp.where(take, s_j - v_k, above)
        found = jnp.where(has, 1, found)
        return (found, jstar, above, carry + s)

    st = lax.fori_loop(0, nv, step, (jnp.int32(0), jnp.int32(0),
                                     jnp.int32(0), jnp.int32(0)))
    return st[1], st[2]


def _sc_body(prob, boxes_in, scale_in,
             boxes_out, labels_out, scores_out,
             hist1, hist2, buf0, buf1, vmaxr, gidx, hitrows, merged, tmp,
             fin_v, fin_i, pv, pi, allv, alli,
             out_s, out_l, out_q, mo_s, mo_l, mo_q,
             boxes_v, bflat, scl,
             sh_h1, sh_h2, sh_fv, sh_fi, sh_os, sh_ol, sh_oq,
             sem0, sem1):
    cc = lax.axis_index("c")
    ss = lax.axis_index("s")
    b = cc * 4 + ss // NW_PER_B
    quarter = ss % NW_PER_B
    sg = (ss // NW_PER_B) * NW_PER_B
    lanes = lax.iota(jnp.int32, 16)
    ones = jnp.ones((16,), jnp.int32)
    zi = jnp.zeros((16,), jnp.int32)
    neg1 = jnp.full((16,), -1, jnp.int32)
    lane_nb1 = lanes * (NB1 + 1)
    lane_nb2 = lanes * (NB2 + 1)

    def zinit(i, _):
        hist1[pl.ds(i * 16, 16)] = zi
        return 0
    lax.fori_loop(0, (16 * (NB1 + 1) + 16) // 16, zinit, 0)

    def zinit2(i, _):
        hist2[pl.ds(i * 16, 16)] = zi
        return 0
    lax.fori_loop(0, (16 * (NB2 + 1) + 16) // 16, zinit2, 0)

    def finit(i, _):
        fin_v[pl.ds(i * 16, 16)] = neg1
        return 0
    lax.fori_loop(0, FCAP // 16, finit, 0)

    def ainit(i, _):
        allv[pl.ds(i * 16, 16)] = neg1
        return 0
    lax.fori_loop(0, ACAP // 16, ainit, 0)

    def oinit(i, _):
        out_s[pl.ds(i * 16, 16)] = zi
        out_l[pl.ds(i * 16, 16)] = zi
        out_q[pl.ds(i * 16, 16)] = zi
        return 0
    lax.fori_loop(0, OSL // 16, oinit, 0)

    qoff = quarter * QN
    pbase = b * FLAT + qoff

    def stream(vec_body, carry):
        cps = {}
        cps[0] = pltpu.async_copy(prob.at[pl.ds(pbase + 0 * CH, CH)],
                                  buf0, sem0)
        for k in range(NCH):
            buf = buf0 if k % 2 == 0 else buf1
            cps[k].wait()
            if k + 1 < NCH:
                cps[k + 1] = pltpu.async_copy(
                    prob.at[pl.ds(pbase + (k + 1) * CH, CH)],
                    buf1 if k % 2 == 0 else buf0,
                    sem1 if k % 2 == 0 else sem0)
            base = qoff + k * CH

            def step(i, cy):
                for u in range(UNR):
                    cy = vec_body(buf, base, i * UNR + u, cy)
                return cy
            carry = lax.fori_loop(0, VPC // UNR, step, carry)
        return carry

    def p1(buf, base, i, cy):
        v = buf[pl.ds(i * 16, 16)]
        key = jnp.minimum((v >> 21) & 0x7FF, NB1 - 1)
        mask = v >= 0
        plsc.addupdate_scatter(hist1, [lane_nb1 + key], ones, mask=mask)
        return cy
    stream(p1, jnp.int32(0))

    def lred1(i, _):
        acc = zi
        for l in range(16):
            acc = acc + hist1[pl.ds(l * (NB1 + 1) + i * 16, 16)]
        merged[pl.ds(i * 16, 16)] = acc
        return 0
    lax.fori_loop(0, NB1 // 16, lred1, 0)
    pltpu.sync_copy(merged.at[pl.ds(0, NB1)], sh_h1.at[pl.ds(ss * NB1, NB1)])
    plsc.subcore_barrier()
    pltpu.sync_copy(sh_h1.at[pl.ds(sg * NB1, NB1)], merged.at[pl.ds(0, NB1)])
    for p in range(1, NW_PER_B):
        pltpu.sync_copy(sh_h1.at[pl.ds((sg + p) * NB1, NB1)], tmp.at[pl.ds(0, NB1)])

        def acc1(i, _):
            merged[pl.ds(i * 16, 16)] = (merged[pl.ds(i * 16, 16)]
                                         + tmp[pl.ds(i * 16, 16)])
            return 0
        lax.fori_loop(0, NB1 // 16, acc1, 0)

    j1, above1 = _suffix_select(merged, jnp.int32(NSEL), lanes, NB1)
    t1 = j1 << 21

    def vinit(i, _):
        vmaxr[pl.ds(i * 16, 16)] = neg1
        return 0
    lax.fori_loop(0, 1008 // 16, vinit, 0)

    def stream2():
        cps = {}
        cps[0] = pltpu.async_copy(prob.at[pl.ds(pbase + 0 * CH, CH)],
                                  buf0, sem0)
        for k in range(NCH):
            buf = buf0 if k % 2 == 0 else buf1
            cps[k].wait()
            if k + 1 < NCH:
                cps[k + 1] = pltpu.async_copy(
                    prob.at[pl.ds(pbase + (k + 1) * CH, CH)],
                    buf1 if k % 2 == 0 else buf0,
                    sem1 if k % 2 == 0 else sem0)
            gbase = k * (VPC // 10)

            def gstep(i, _):
                m = None
                for u in range(10):
                    v = buf[pl.ds((i * 10 + u) * 16, 16)]
                    mask = (v >> 21) == j1
                    key2 = (v >> 11) & (NB2 - 1)
                    plsc.addupdate_scatter(hist2, [lane_nb2 + key2], ones,
                                           mask=mask)
                    m = v if u == 0 else jnp.maximum(m, v)
                gm = lax.reduce_max_p.bind(m, axes=(0,))
                plsc.store_scatter(
                    vmaxr, [jnp.full((16,), gbase + i, jnp.int32)],
                    jnp.full((16,), gm, jnp.int32), mask=lanes == 0)
                return 0
            lax.fori_loop(0, VPC // 10, gstep, 0)
    stream2()

    def lred2(i, _):
        acc = zi
        for l in range(16):
            acc = acc + hist2[pl.ds(l * (NB2 + 1) + i * 16, 16)]
        merged[pl.ds(i * 16, 16)] = acc
        return 0
    lax.fori_loop(0, NB2 // 16, lred2, 0)
    pltpu.sync_copy(merged, sh_h2.at[pl.ds(ss * NB2, NB2)])
    plsc.subcore_barrier()
    pltpu.sync_copy(sh_h2.at[pl.ds(sg * NB2, NB2)], merged)
    for p in range(1, NW_PER_B):
        pltpu.sync_copy(sh_h2.at[pl.ds((sg + p) * NB2, NB2)], tmp)

        def acc2(i, _):
            merged[pl.ds(i * 16, 16)] = (merged[pl.ds(i * 16, 16)]
                                         + tmp[pl.ds(i * 16, 16)])
            return 0
        lax.fori_loop(0, NB2 // 16, acc2, 0)

    j2, _ = _suffix_select(merged, jnp.int32(NSEL) - above1, lanes, NB2)
    t2 = t1 | (j2 << 11)

    def ginit(i, _):
        gidx[pl.ds(i * 16, 16)] = zi
        return 0
    lax.fori_loop(0, GCAP // 16, ginit, 0)

    def scanv(i, cy):
        m = vmaxr[pl.ds(i * 16, 16)]
        mask = m >= t2
        cs = plsc.cumsum(mask.astype(jnp.int32))
        pos = jnp.minimum(cy + cs - 1, GCAP - 1)
        g = i * 16 + lanes
        plsc.store_scatter(gidx, [pos], g, mask=mask)
        return cy + lax.reduce_sum_p.bind(jnp.where(mask, 1, 0), axes=(0,))
    nhit = lax.fori_loop(0, 1008 // 16, scanv, jnp.int32(0))

    nblk = (nhit + 3) >> 2

    def fblk(bi, cy):
        gs = []
        cps = []
        for u in range(4):
            r = bi * 4 + u
            gvec = gidx[pl.ds((r >> 4) * 16, 16)]
            oneh = lanes == (r & 15)
            g_r = lax.reduce_sum_p.bind(jnp.where(oneh, gvec, 0), axes=(0,))
            gs.append(g_r)
            cps.append(pltpu.async_copy(
                prob.at[pl.ds(pbase + g_r * 160, 160)],
                hitrows.at[pl.ds(u * 160, 160)], sem0))
        for cp in cps:
            cp.wait()
        for u in range(4):
            r = bi * 4 + u
            for j in range(10):
                v = hitrows[pl.ds(u * 160 + j * 16, 16)]
                off = qoff + gs[u] * 160 + j * 16 + lanes
                mask = jnp.logical_and(v >= t2, r < nhit)
                cs = plsc.cumsum(mask.astype(jnp.int32))
                pos = jnp.minimum(cy + cs - 1, FCAP - 1)
                plsc.store_scatter(fin_v, [pos], v, mask=mask)
                plsc.store_scatter(fin_i, [pos], off, mask=mask)
                cy = cy + lax.reduce_sum_p.bind(jnp.where(mask, 1, 0),
                                                axes=(0,))
        return cy
    fcnt = lax.fori_loop(0, nblk, fblk, jnp.int32(0))

    pltpu.sync_copy(fin_v, sh_fv.at[pl.ds(ss * FCAP, FCAP)])
    pltpu.sync_copy(fin_i, sh_fi.at[pl.ds(ss * FCAP, FCAP)])
    plsc.subcore_barrier()

    mtot = jnp.int32(0)
    for p in range(NW_PER_B):
        pltpu.sync_copy(sh_fv.at[pl.ds((sg + p) * FCAP, FCAP)], pv)
        pltpu.sync_copy(sh_fi.at[pl.ds((sg + p) * FCAP, FCAP)], pi)

        def gcomp(i, cy):
            v = pv[pl.ds(i * 16, 16)]
            iv = pi[pl.ds(i * 16, 16)]
            mask = v >= 0
            cs = plsc.cumsum(mask.astype(jnp.int32))
            pos = jnp.minimum(cy + cs - 1, ACAP - 1)
            plsc.store_scatter(allv, [pos], v, mask=mask)
            plsc.store_scatter(alli, [pos], iv, mask=mask)
            return cy + lax.reduce_sum_p.bind(jnp.where(mask, 1, 0),
                                              axes=(0,))
        mtot = lax.fori_loop(0, FCAP // 16, gcomp, mtot)

    nj = (mtot + 15) >> 4

    def rank_outer(iv, _):
        @pl.when(iv * 16 < fcnt)
        def _():
            ov = fin_v[pl.ds(iv * 16, 16)]
            oi = fin_i[pl.ds(iv * 16, 16)]
            for ii in range(16):
                oneh = lanes == ii
                v_i = lax.reduce_sum_p.bind(jnp.where(oneh, ov, 0),
                                            axes=(0,))
                off_i = lax.reduce_sum_p.bind(jnp.where(oneh, oi, 0),
                                              axes=(0,))

                def jstep(jj, racc):
                    vj = allv[pl.ds(jj * 16, 16)]
                    ij = alli[pl.ds(jj * 16, 16)]
                    gt = vj > v_i
                    eq = jnp.logical_and(vj == v_i, ij < off_i)
                    return racc + jnp.where(jnp.logical_or(gt, eq), 1, 0)

                racc = lax.fori_loop(0, nj, jstep, zi)
                r_i = lax.reduce_sum_p.bind(racc, axes=(0,))

                @pl.when(jnp.logical_and(iv * 16 + ii < fcnt, r_i < NSEL))
                def _():
                    lane0 = lanes == 0
                    ridx = jnp.full((16,), r_i, jnp.int32)
                    plsc.store_scatter(out_s, [ridx],
                                       jnp.full((16,), v_i, jnp.int32),
                                       mask=lane0)
                    plsc.store_scatter(out_l, [ridx],
                                       jnp.full((16,), off_i & (PC - 1),
                                                jnp.int32), mask=lane0)
                    plsc.store_scatter(out_q, [ridx],
                                       jnp.full((16,), off_i >> 7,
                                                jnp.int32), mask=lane0)
        return 0
    lax.fori_loop(0, FCAP // 16, rank_outer, 0)

    pltpu.sync_copy(out_s, sh_os.at[pl.ds(ss * OSL, OSL)])
    pltpu.sync_copy(out_l, sh_ol.at[pl.ds(ss * OSL, OSL)])
    pltpu.sync_copy(out_q, sh_oq.at[pl.ds(ss * OSL, OSL)])
    plsc.subcore_barrier()

    @pl.when(quarter == 0)
    def _():
        pltpu.sync_copy(sh_os.at[pl.ds(sg * OSL, OSL)], mo_s)
        pltpu.sync_copy(sh_ol.at[pl.ds(sg * OSL, OSL)], mo_l)
        pltpu.sync_copy(sh_oq.at[pl.ds(sg * OSL, OSL)], mo_q)
        for p in range(1, NW_PER_B):
            pltpu.sync_copy(sh_os.at[pl.ds((sg + p) * OSL, OSL)], out_s)
            pltpu.sync_copy(sh_ol.at[pl.ds((sg + p) * OSL, OSL)], out_l)
            pltpu.sync_copy(sh_oq.at[pl.ds((sg + p) * OSL, OSL)], out_q)

            def omerge(i, _):
                sl = pl.ds(i * 16, 16)
                mo_s[sl] = mo_s[sl] + out_s[sl]
                mo_l[sl] = mo_l[sl] + out_l[sl]
                mo_q[sl] = mo_q[sl] + out_q[sl]
                return 0
            lax.fori_loop(0, OSL // 16, omerge, 0)

        pltpu.sync_copy(boxes_in.at[pl.ds(b * N * 4, N * 4)], boxes_v)
        pltpu.sync_copy(scale_in.at[pl.ds(b * 16, 16)], scl)
        sv = scl[...]
        s0 = lax.reduce_sum_p.bind(jnp.where(lanes == 0, sv, 0.0), axes=(0,))
        s1 = lax.reduce_sum_p.bind(jnp.where(lanes == 1, sv, 0.0), axes=(0,))
        s2 = lax.reduce_sum_p.bind(jnp.where(lanes == 2, sv, 0.0), axes=(0,))
        s3 = lax.reduce_sum_p.bind(jnp.where(lanes == 3, sv, 0.0), axes=(0,))

        def bgather(k, _):
            qv4 = mo_q[pl.ds(k * 16, 16)] * 4
            cx = plsc.load_gather(boxes_v, [qv4])
            cyv = plsc.load_gather(boxes_v, [qv4 + 1])
            w = plsc.load_gather(boxes_v, [qv4 + 2])
            h = plsc.load_gather(boxes_v, [qv4 + 3])
            x = (cx - 0.5 * w) * s0
            y = (cyv - 0.5 * h) * s1
            ws = w * s2
            hs = h * s3
            slot4 = (k * 16 + lanes) * 4
            plsc.store_scatter(bflat, [slot4], x)
            plsc.store_scatter(bflat, [slot4 + 1], y)
            plsc.store_scatter(bflat, [slot4 + 2], ws)
            plsc.store_scatter(bflat, [slot4 + 3], hs)
            return 0
        lax.fori_loop(0, OSL // 16, bgather, 0)

        pltpu.sync_copy(bflat, boxes_out.at[pl.ds(b * OSL * 4, OSL * 4)])
        pltpu.sync_copy(mo_s, scores_out.at[pl.ds(b * OSL, OSL)])
        pltpu.sync_copy(mo_l, labels_out.at[pl.ds(b * OSL, OSL)])


def _make_sc_kernel():
    mesh = plsc.VectorSubcoreMesh(core_axis_name="c", subcore_axis_name="s")
    return pl.kernel(
        _sc_body,
        out_type=(
            jax.ShapeDtypeStruct((B * OSL * 4,), jnp.float32),
            jax.ShapeDtypeStruct((B * OSL,), jnp.int32),
            jax.ShapeDtypeStruct((B * OSL,), jnp.int32),
        ),
        mesh=mesh,
        compiler_params=pltpu.CompilerParams(needs_layout_passes=False),
        scratch_types=[
            pltpu.VMEM((16 * (NB1 + 1) + 16,), jnp.int32),
            pltpu.VMEM((16 * (NB2 + 1) + 16,), jnp.int32),
            pltpu.VMEM((CH,), jnp.int32),
            pltpu.VMEM((CH,), jnp.int32),
            pltpu.VMEM((1008,), jnp.int32),
            pltpu.VMEM((GCAP,), jnp.int32),
            pltpu.VMEM((640,), jnp.int32),
            pltpu.VMEM((NB2,), jnp.int32),
            pltpu.VMEM((NB2,), jnp.int32),
            pltpu.VMEM((FCAP,), jnp.int32),
            pltpu.VMEM((FCAP,), jnp.int32),
            pltpu.VMEM((FCAP,), jnp.int32),
            pltpu.VMEM((FCAP,), jnp.int32),
            pltpu.VMEM((ACAP,), jnp.int32),
            pltpu.VMEM((ACAP,), jnp.int32),
            pltpu.VMEM((OSL,), jnp.int32),
            pltpu.VMEM((OSL,), jnp.int32),
            pltpu.VMEM((OSL,), jnp.int32),
            pltpu.VMEM((OSL,), jnp.int32),
            pltpu.VMEM((OSL,), jnp.int32),
            pltpu.VMEM((OSL,), jnp.int32),
            pltpu.VMEM((N * 4,), jnp.float32),
            pltpu.VMEM((OSL * 4,), jnp.float32),
            pltpu.VMEM((16,), jnp.float32),
            pltpu.VMEM_SHARED((16 * NB1,), jnp.int32),
            pltpu.VMEM_SHARED((16 * NB2,), jnp.int32),
            pltpu.VMEM_SHARED((16 * FCAP,), jnp.int32),
            pltpu.VMEM_SHARED((16 * FCAP,), jnp.int32),
            pltpu.VMEM_SHARED((16 * OSL,), jnp.int32),
            pltpu.VMEM_SHARED((16 * OSL,), jnp.int32),
            pltpu.VMEM_SHARED((16 * OSL,), jnp.int32),
            pltpu.SemaphoreType.DMA,
            pltpu.SemaphoreType.DMA,
        ],
    )


def kernel(pred_logits, pred_boxes, target_sizes, positive_map):
    wT = jnp.pad(positive_map, ((0, PC - C), (0, 0))).T
    NBLK = 1000
    prob = pl.pallas_call(
        _mm_kernel,
        grid=(B, N // NBLK),
        in_specs=[
            pl.BlockSpec((1, NBLK, T), lambda i, j: (i, j, 0)),
            pl.BlockSpec((T, PC), lambda i, j: (0, 0)),
        ],
        out_specs=pl.BlockSpec((1, NBLK, PC), lambda i, j: (i, j, 0)),
        out_shape=jax.ShapeDtypeStruct((B, N, PC), jnp.int32),
    )(pred_logits, wT)

    img_h = target_sizes[:, 0]
    img_w = target_sizes[:, 1]
    scale = jnp.stack([img_w, img_h, img_w, img_h], axis=1).astype(jnp.float32)
    scale16 = jnp.pad(scale, ((0, 0), (0, 12)))

    boxes_flat, labels_p, scores_p = _make_sc_kernel()(
        prob.reshape(-1), pred_boxes.reshape(-1), scale16.reshape(-1))
    boxes = boxes_flat.reshape(B, OSL, 4)[:, :NSEL, :]
    labels = labels_p.reshape(B, OSL)[:, :NSEL]
    scores = lax.bitcast_convert_type(
        scores_p.reshape(B, OSL)[:, :NSEL], jnp.float32)
    return (boxes, labels, scores)


# speedup vs baseline: 1.3706x; 1.0006x over previous
import jax
import jax.numpy as jnp
from jax import lax
from jax.experimental import pallas as pl
from jax.experimental.pallas import tpu as pltpu
from jax.experimental.pallas import tpu_sc as plsc

NSEL = 300
B = 8
N = 5000
T = 256
C = 81
PC = 128
FLAT = N * PC
NW_PER_B = 4
QN = FLAT // NW_PER_B
CH = 8000
NCH = QN // CH
VPC = CH // 16
UNR = 10
NB1 = 1024
NB2 = 1024
NVG = QN // 160
GCAP = 512
FCAP = 512
ACAP = 2048
OSL = 304


def _mm_kernel(x_ref, w_ref, o_ref):
    s = jax.nn.sigmoid(x_ref[0])
    y = jnp.dot(s, w_ref[...], preferred_element_type=jnp.float32)
    col = lax.broadcasted_iota(jnp.int32, y.shape, 1)
    o_ref[0] = lax.bitcast_convert_type(
        jnp.where(col < C, y, -1.0), jnp.int32)


def _suffix_select(merged, target, lanes, nb):
    nv = nb // 16

    def step(t, st):
        found, jstar, above, carry = st
        j = nv - 1 - t
        vec = merged[pl.ds(j * 16, 16)]
        c = plsc.cumsum(vec)
        s = lax.reduce_sum_p.bind(vec, axes=(0,))
        suffix = carry + (s - c + vec)
        cond = (suffix >= target).astype(jnp.int32)
        pc = lax.reduce_sum_p.bind(cond, axes=(0,))
        has = pc > 0
        kstar = pc - 1
        oneh = lanes == kstar
        c_k = lax.reduce_sum_p.bind(jnp.where(oneh, c, 0), axes=(0,))
        v_k = lax.reduce_sum_p.bind(jnp.where(oneh, vec, 0), axes=(0,))
        s_j = carry + s - c_k + v_k
        take = jnp.logical_and(found == 0, has)
        jstar = jnp.where(take, j * 16 + kstar, jstar)
        above = jnp.where(take, s_j - v_k, above)
        found = jnp.where(has, 1, found)
        return (found, jstar, above, carry + s)

    st = lax.fori_loop(0, nv, step, (jnp.int32(0), jnp.int32(0),
                                     jnp.int32(0), jnp.int32(0)))
    return st[1], st[2]


def _sc_body(prob, boxes_in, scale_in,
             boxes_out, labels_out, scores_out,
             hist1, hist2, buf0, buf1, vmaxr, gidx, hitrows, merged, tmp,
             fin_v, fin_i, pv, pi, allv, alli,
             out_s, out_l, out_q, mo_s, mo_l, mo_q,
             boxes_v, bflat, scl,
             sh_h1, sh_h2, sh_fv, sh_fi, sh_os, sh_ol, sh_oq,
             sem0, sem1):
    cc = lax.axis_index("c")
    ss = lax.axis_index("s")
    b = cc * 4 + ss // NW_PER_B
    quarter = ss % NW_PER_B
    sg = (ss // NW_PER_B) * NW_PER_B
    lanes = lax.iota(jnp.int32, 16)
    ones = jnp.ones((16,), jnp.int32)
    zi = jnp.zeros((16,), jnp.int32)
    neg1 = jnp.full((16,), -1, jnp.int32)
    lane_nb1 = lanes * (NB1 + 1)
    lane_nb2 = lanes * (NB2 + 1)

    def zinit(i, _):
        hist1[pl.ds(i * 16, 16)] = zi
        return 0
    lax.fori_loop(0, (16 * (NB1 + 1) + 16) // 16, zinit, 0)

    def zinit2(i, _):
        hist2[pl.ds(i * 16, 16)] = zi
        return 0
    lax.fori_loop(0, (16 * (NB2 + 1) + 16) // 16, zinit2, 0)

    def finit(i, _):
        fin_v[pl.ds(i * 16, 16)] = neg1
        return 0
    lax.fori_loop(0, FCAP // 16, finit, 0)

    def ainit(i, _):
        allv[pl.ds(i * 16, 16)] = neg1
        return 0
    lax.fori_loop(0, ACAP // 16, ainit, 0)

    def oinit(i, _):
        out_s[pl.ds(i * 16, 16)] = zi
        out_l[pl.ds(i * 16, 16)] = zi
        out_q[pl.ds(i * 16, 16)] = zi
        return 0
    lax.fori_loop(0, OSL // 16, oinit, 0)

    qoff = quarter * QN
    pbase = b * FLAT + qoff

    def stream(vec_body, carry):
        cps = {}
        cps[0] = pltpu.async_copy(prob.at[pl.ds(pbase + 0 * CH, CH)],
                                  buf0, sem0)
        for k in range(NCH):
            buf = buf0 if k % 2 == 0 else buf1
            cps[k].wait()
            if k + 1 < NCH:
                cps[k + 1] = pltpu.async_copy(
                    prob.at[pl.ds(pbase + (k + 1) * CH, CH)],
                    buf1 if k % 2 == 0 else buf0,
                    sem1 if k % 2 == 0 else sem0)
            base = qoff + k * CH

            def step(i, cy):
                for u in range(UNR):
                    cy = vec_body(buf, base, i * UNR + u, cy, u)
                return cy
            carry = lax.fori_loop(0, VPC // UNR, step, carry)
        return carry

    rot1 = [((lanes + u) & 15) * (NB1 + 1) for u in range(UNR)]

    def p1(buf, base, i, cy, u):
        v = buf[pl.ds(i * 16, 16)]
        key = jnp.minimum((v >> 21) & 0x7FF, NB1 - 1)
        mask = v >= 0
        plsc.addupdate_scatter(hist1, [rot1[u] + key], ones, mask=mask)
        return cy
    stream(p1, jnp.int32(0))

    def lred1(i, _):
        acc = zi
        for l in range(16):
            acc = acc + hist1[pl.ds(l * (NB1 + 1) + i * 16, 16)]
        merged[pl.ds(i * 16, 16)] = acc
        return 0
    lax.fori_loop(0, NB1 // 16, lred1, 0)
    pltpu.sync_copy(merged.at[pl.ds(0, NB1)], sh_h1.at[pl.ds(ss * NB1, NB1)])
    plsc.subcore_barrier()
    pltpu.sync_copy(sh_h1.at[pl.ds(sg * NB1, NB1)], merged.at[pl.ds(0, NB1)])
    for p in range(1, NW_PER_B):
        pltpu.sync_copy(sh_h1.at[pl.ds((sg + p) * NB1, NB1)], tmp.at[pl.ds(0, NB1)])

        def acc1(i, _):
            merged[pl.ds(i * 16, 16)] = (merged[pl.ds(i * 16, 16)]
                                         + tmp[pl.ds(i * 16, 16)])
            return 0
        lax.fori_loop(0, NB1 // 16, acc1, 0)

    j1, above1 = _suffix_select(merged, jnp.int32(NSEL), lanes, NB1)
    t1 = j1 << 21

    def vinit(i, _):
        vmaxr[pl.ds(i * 16, 16)] = neg1
        return 0
    lax.fori_loop(0, 1008 // 16, vinit, 0)

    rot2 = [((lanes + u) & 15) * (NB2 + 1) for u in range(10)]

    def stream2():
        cps = {}
        cps[0] = pltpu.async_copy(prob.at[pl.ds(pbase + 0 * CH, CH)],
                                  buf0, sem0)
        for k in range(NCH):
            buf = buf0 if k % 2 == 0 else buf1
            cps[k].wait()
            if k + 1 < NCH:
                cps[k + 1] = pltpu.async_copy(
                    prob.at[pl.ds(pbase + (k + 1) * CH, CH)],
                    buf1 if k % 2 == 0 else buf0,
                    sem1 if k % 2 == 0 else sem0)
            gbase = k * (VPC // 10)

            def gstep(i, _):
                m = None
                for u in range(10):
                    v = buf[pl.ds((i * 10 + u) * 16, 16)]
                    mask = (v >> 21) == j1
                    key2 = (v >> 11) & (NB2 - 1)
                    plsc.addupdate_scatter(hist2, [rot2[u] + key2], ones,
                                           mask=mask)
                    m = v if u == 0 else jnp.maximum(m, v)
                gm = lax.reduce_max_p.bind(m, axes=(0,))
                plsc.store_scatter(
                    vmaxr, [jnp.full((16,), gbase + i, jnp.int32)],
                    jnp.full((16,), gm, jnp.int32), mask=lanes == 0)
                return 0
            lax.fori_loop(0, VPC // 10, gstep, 0)
    stream2()

    def lred2(i, _):
        acc = zi
        for l in range(16):
            acc = acc + hist2[pl.ds(l * (NB2 + 1) + i * 16, 16)]
        merged[pl.ds(i * 16, 16)] = acc
        return 0
    lax.fori_loop(0, NB2 // 16, lred2, 0)
    pltpu.sync_copy(merged, sh_h2.at[pl.ds(ss * NB2, NB2)])
    plsc.subcore_barrier()
    pltpu.sync_copy(sh_h2.at[pl.ds(sg * NB2, NB2)], merged)
    for p in range(1, NW_PER_B):
        pltpu.sync_copy(sh_h2.at[pl.ds((sg + p) * NB2, NB2)], tmp)

        def acc2(i, _):
            merged[pl.ds(i * 16, 16)] = (merged[pl.ds(i * 16, 16)]
                                         + tmp[pl.ds(i * 16, 16)])
            return 0
        lax.fori_loop(0, NB2 // 16, acc2, 0)

    j2, _ = _suffix_select(merged, jnp.int32(NSEL) - above1, lanes, NB2)
    t2 = t1 | (j2 << 11)

    def ginit(i, _):
        gidx[pl.ds(i * 16, 16)] = zi
        return 0
    lax.fori_loop(0, GCAP // 16, ginit, 0)

    def scanv(i, cy):
        m = vmaxr[pl.ds(i * 16, 16)]
        mask = m >= t2
        cs = plsc.cumsum(mask.astype(jnp.int32))
        pos = jnp.minimum(cy + cs - 1, GCAP - 1)
        g = i * 16 + lanes
        plsc.store_scatter(gidx, [pos], g, mask=mask)
        return cy + lax.reduce_sum_p.bind(jnp.where(mask, 1, 0), axes=(0,))
    nhit = lax.fori_loop(0, 1008 // 16, scanv, jnp.int32(0))

    nblk = (nhit + 3) >> 2

    def fblk(bi, cy):
        gs = []
        cps = []
        for u in range(4):
            r = bi * 4 + u
            gvec = gidx[pl.ds((r >> 4) * 16, 16)]
            oneh = lanes == (r & 15)
            g_r = lax.reduce_sum_p.bind(jnp.where(oneh, gvec, 0), axes=(0,))
            gs.append(g_r)
            cps.append(pltpu.async_copy(
                prob.at[pl.ds(pbase + g_r * 160, 160)],
                hitrows.at[pl.ds(u * 160, 160)], sem0))
        for cp in cps:
            cp.wait()
        for u in range(4):
            r = bi * 4 + u
            for j in range(10):
                v = hitrows[pl.ds(u * 160 + j * 16, 16)]
                off = qoff + gs[u] * 160 + j * 16 + lanes
                mask = jnp.logical_and(v >= t2, r < nhit)
                cs = plsc.cumsum(mask.astype(jnp.int32))
                pos = jnp.minimum(cy + cs - 1, FCAP - 1)
                plsc.store_scatter(fin_v, [pos], v, mask=mask)
                plsc.store_scatter(fin_i, [pos], off, mask=mask)
                cy = cy + lax.reduce_sum_p.bind(jnp.where(mask, 1, 0),
                                                axes=(0,))
        return cy
    fcnt = lax.fori_loop(0, nblk, fblk, jnp.int32(0))

    pltpu.sync_copy(fin_v, sh_fv.at[pl.ds(ss * FCAP, FCAP)])
    pltpu.sync_copy(fin_i, sh_fi.at[pl.ds(ss * FCAP, FCAP)])
    plsc.subcore_barrier()

    mtot = jnp.int32(0)
    for p in range(NW_PER_B):
        pltpu.sync_copy(sh_fv.at[pl.ds((sg + p) * FCAP, FCAP)], pv)
        pltpu.sync_copy(sh_fi.at[pl.ds((sg + p) * FCAP, FCAP)], pi)

        def gcomp(i, cy):
            v = pv[pl.ds(i * 16, 16)]
            iv = pi[pl.ds(i * 16, 16)]
            mask = v >= 0
            cs = plsc.cumsum(mask.astype(jnp.int32))
            pos = jnp.minimum(cy + cs - 1, ACAP - 1)
            plsc.store_scatter(allv, [pos], v, mask=mask)
            plsc.store_scatter(alli, [pos], iv, mask=mask)
            return cy + lax.reduce_sum_p.bind(jnp.where(mask, 1, 0),
                                              axes=(0,))
        mtot = lax.fori_loop(0, FCAP // 16, gcomp, mtot)

    nj = (mtot + 15) >> 4

    def rank_outer(iv, _):
        @pl.when(iv * 16 < fcnt)
        def _():
            ov = fin_v[pl.ds(iv * 16, 16)]
            oi = fin_i[pl.ds(iv * 16, 16)]
            for ii in range(16):
                oneh = lanes == ii
                v_i = lax.reduce_sum_p.bind(jnp.where(oneh, ov, 0),
                                            axes=(0,))
                off_i = lax.reduce_sum_p.bind(jnp.where(oneh, oi, 0),
                                              axes=(0,))

                def jstep(jj, racc):
                    vj = allv[pl.ds(jj * 16, 16)]
                    ij = alli[pl.ds(jj * 16, 16)]
                    gt = vj > v_i
                    eq = jnp.logical_and(vj == v_i, ij < off_i)
                    return racc + jnp.where(jnp.logical_or(gt, eq), 1, 0)

                racc = lax.fori_loop(0, nj, jstep, zi)
                r_i = lax.reduce_sum_p.bind(racc, axes=(0,))

                @pl.when(jnp.logical_and(iv * 16 + ii < fcnt, r_i < NSEL))
                def _():
                    lane0 = lanes == 0
                    ridx = jnp.full((16,), r_i, jnp.int32)
                    plsc.store_scatter(out_s, [ridx],
                                       jnp.full((16,), v_i, jnp.int32),
                                       mask=lane0)
                    plsc.store_scatter(out_l, [ridx],
                                       jnp.full((16,), off_i & (PC - 1),
                                                jnp.int32), mask=lane0)
                    plsc.store_scatter(out_q, [ridx],
                                       jnp.full((16,), off_i >> 7,
                                                jnp.int32), mask=lane0)
        return 0
    lax.fori_loop(0, FCAP // 16, rank_outer, 0)

    pltpu.sync_copy(out_s, sh_os.at[pl.ds(ss * OSL, OSL)])
    pltpu.sync_copy(out_l, sh_ol.at[pl.ds(ss * OSL, OSL)])
    pltpu.sync_copy(out_q, sh_oq.at[pl.ds(ss * OSL, OSL)])
    plsc.subcore_barrier()

    @pl.when(quarter == 0)
    def _():
        pltpu.sync_copy(sh_os.at[pl.ds(sg * OSL, OSL)], mo_s)
        pltpu.sync_copy(sh_ol.at[pl.ds(sg * OSL, OSL)], mo_l)
        pltpu.sync_copy(sh_oq.at[pl.ds(sg * OSL, OSL)], mo_q)
        for p in range(1, NW_PER_B):
            pltpu.sync_copy(sh_os.at[pl.ds((sg + p) * OSL, OSL)], out_s)
            pltpu.sync_copy(sh_ol.at[pl.ds((sg + p) * OSL, OSL)], out_l)
            pltpu.sync_copy(sh_oq.at[pl.ds((sg + p) * OSL, OSL)], out_q)

            def omerge(i, _):
                sl = pl.ds(i * 16, 16)
                mo_s[sl] = mo_s[sl] + out_s[sl]
                mo_l[sl] = mo_l[sl] + out_l[sl]
                mo_q[sl] = mo_q[sl] + out_q[sl]
                return 0
            lax.fori_loop(0, OSL // 16, omerge, 0)

        pltpu.sync_copy(boxes_in.at[pl.ds(b * N * 4, N * 4)], boxes_v)
        pltpu.sync_copy(scale_in.at[pl.ds(b * 16, 16)], scl)
        sv = scl[...]
        s0 = lax.reduce_sum_p.bind(jnp.where(lanes == 0, sv, 0.0), axes=(0,))
        s1 = lax.reduce_sum_p.bind(jnp.where(lanes == 1, sv, 0.0), axes=(0,))
        s2 = lax.reduce_sum_p.bind(jnp.where(lanes == 2, sv, 0.0), axes=(0,))
        s3 = lax.reduce_sum_p.bind(jnp.where(lanes == 3, sv, 0.0), axes=(0,))

        def bgather(k, _):
            qv4 = mo_q[pl.ds(k * 16, 16)] * 4
            cx = plsc.load_gather(boxes_v, [qv4])
            cyv = plsc.load_gather(boxes_v, [qv4 + 1])
            w = plsc.load_gather(boxes_v, [qv4 + 2])
            h = plsc.load_gather(boxes_v, [qv4 + 3])
            x = (cx - 0.5 * w) * s0
            y = (cyv - 0.5 * h) * s1
            ws = w * s2
            hs = h * s3
            slot4 = (k * 16 + lanes) * 4
            plsc.store_scatter(bflat, [slot4], x)
            plsc.store_scatter(bflat, [slot4 + 1], y)
            plsc.store_scatter(bflat, [slot4 + 2], ws)
            plsc.store_scatter(bflat, [slot4 + 3], hs)
            return 0
        lax.fori_loop(0, OSL // 16, bgather, 0)

        pltpu.sync_copy(bflat, boxes_out.at[pl.ds(b * OSL * 4, OSL * 4)])
        pltpu.sync_copy(mo_s, scores_out.at[pl.ds(b * OSL, OSL)])
        pltpu.sync_copy(mo_l, labels_out.at[pl.ds(b * OSL, OSL)])


def _make_sc_kernel():
    mesh = plsc.VectorSubcoreMesh(core_axis_name="c", subcore_axis_name="s")
    return pl.kernel(
        _sc_body,
        out_type=(
            jax.ShapeDtypeStruct((B * OSL * 4,), jnp.float32),
            jax.ShapeDtypeStruct((B * OSL,), jnp.int32),
            jax.ShapeDtypeStruct((B * OSL,), jnp.int32),
        ),
        mesh=mesh,
        compiler_params=pltpu.CompilerParams(needs_layout_passes=False),
        scratch_types=[
            pltpu.VMEM((16 * (NB1 + 1) + 16,), jnp.int32),
            pltpu.VMEM((16 * (NB2 + 1) + 16,), jnp.int32),
            pltpu.VMEM((CH,), jnp.int32),
            pltpu.VMEM((CH,), jnp.int32),
            pltpu.VMEM((1008,), jnp.int32),
            pltpu.VMEM((GCAP,), jnp.int32),
            pltpu.VMEM((640,), jnp.int32),
            pltpu.VMEM((NB2,), jnp.int32),
            pltpu.VMEM((NB2,), jnp.int32),
            pltpu.VMEM((FCAP,), jnp.int32),
            pltpu.VMEM((FCAP,), jnp.int32),
            pltpu.VMEM((FCAP,), jnp.int32),
            pltpu.VMEM((FCAP,), jnp.int32),
            pltpu.VMEM((ACAP,), jnp.int32),
            pltpu.VMEM((ACAP,), jnp.int32),
            pltpu.VMEM((OSL,), jnp.int32),
            pltpu.VMEM((OSL,), jnp.int32),
            pltpu.VMEM((OSL,), jnp.int32),
            pltpu.VMEM((OSL,), jnp.int32),
            pltpu.VMEM((OSL,), jnp.int32),
            pltpu.VMEM((OSL,), jnp.int32),
            pltpu.VMEM((N * 4,), jnp.float32),
            pltpu.VMEM((OSL * 4,), jnp.float32),
            pltpu.VMEM((16,), jnp.float32),
            pltpu.VMEM_SHARED((16 * NB1,), jnp.int32),
            pltpu.VMEM_SHARED((16 * NB2,), jnp.int32),
            pltpu.VMEM_SHARED((16 * FCAP,), jnp.int32),
            pltpu.VMEM_SHARED((16 * FCAP,), jnp.int32),
            pltpu.VMEM_SHARED((16 * OSL,), jnp.int32),
            pltpu.VMEM_SHARED((16 * OSL,), jnp.int32),
            pltpu.VMEM_SHARED((16 * OSL,), jnp.int32),
            pltpu.SemaphoreType.DMA,
            pltpu.SemaphoreType.DMA,
        ],
    )


def kernel(pred_logits, pred_boxes, target_sizes, positive_map):
    wT = jnp.pad(positive_map, ((0, PC - C), (0, 0))).T
    NBLK = 1000
    prob = pl.pallas_call(
        _mm_kernel,
        grid=(B, N // NBLK),
        in_specs=[
            pl.BlockSpec((1, NBLK, T), lambda i, j: (i, j, 0)),
            pl.BlockSpec((T, PC), lambda i, j: (0, 0)),
        ],
        out_specs=pl.BlockSpec((1, NBLK, PC), lambda i, j: (i, j, 0)),
        out_shape=jax.ShapeDtypeStruct((B, N, PC), jnp.int32),
    )(pred_logits, wT)

    img_h = target_sizes[:, 0]
    img_w = target_sizes[:, 1]
    scale = jnp.stack([img_w, img_h, img_w, img_h], axis=1).astype(jnp.float32)
    scale16 = jnp.pad(scale, ((0, 0), (0, 12)))

    boxes_flat, labels_p, scores_p = _make_sc_kernel()(
        prob.reshape(-1), pred_boxes.reshape(-1), scale16.reshape(-1))
    boxes = boxes_flat.reshape(B, OSL, 4)[:, :NSEL, :]
    labels = labels_p.reshape(B, OSL)[:, :NSEL]
    scores = lax.bitcast_convert_type(
        scores_p.reshape(B, OSL)[:, :NSEL], jnp.float32)
    return (boxes, labels, scores)
